# Initial kernel scaffold; baseline (speedup 1.0000x reference)
import functools

import jax
import jax.numpy as jnp
from jax import lax
from jax.experimental import pallas as pl
from jax.experimental.pallas import tpu as pltpu
from jax.experimental.pallas import tpu_sc as plsc

N = 10000
E = 320000
IN_CH = 256
OUT_CH = 256
POS = 8
HALF = OUT_CH // 2

NC = 2
NS = 16
L = 16

CHUNK = 128
R_PAD = 10240
C_T = 2528
E_PAD = C_T * CHUNK
CB = C_T // (NC * NS)
CD = C_T // NS
RT = R_PAD // NS

BR = 1000
BRP = 1024


def _proj_body(x_ref, pos_ref, wp_ref, wc_ref, y_ref):
    x = x_ref[...]
    pos = pos_ref[...]
    acc = jnp.zeros((BR, OUT_CH), jnp.float32)
    for p in range(POS):
        t = lax.dot_general(x, wp_ref[p * OUT_CH:(p + 1) * OUT_CH, :],
                            (((1,), (1,)), ((), ())),
                            preferred_element_type=jnp.float32)
        acc = acc + pos[:, p:p + 1] * t
    y_ref[...] = lax.dot_general(acc, wc_ref[...],
                                 (((1,), (1,)), ((), ())),
                                 preferred_element_type=jnp.float32)


def _proj(x, pos, Wp, Wc):
    return pl.pallas_call(
        _proj_body,
        grid=(N // BR,),
        in_specs=[
            pl.BlockSpec((BR, IN_CH), lambda i: (i, 0)),
            pl.BlockSpec((BR, POS), lambda i: (i, 0)),
            pl.BlockSpec((POS * OUT_CH, IN_CH), lambda i: (0, 0)),
            pl.BlockSpec((OUT_CH, OUT_CH), lambda i: (0, 0)),
        ],
        out_specs=pl.BlockSpec((BR, OUT_CH), lambda i: (i, 0)),
        out_shape=jax.ShapeDtypeStruct((N, OUT_CH), jnp.float32),
    )(x, pos, Wp, Wc)


def _scale_body(y_ref, degT_ref, hn0_ref, hn1_ref):
    deg = degT_ref[:, 0:1] + degT_ref[:, 1:2] + 1.0
    dinv = lax.rsqrt(deg)
    hn = y_ref[...] * dinv
    hn0_ref[...] = hn[:, :HALF]
    hn1_ref[...] = hn[:, HALF:]


def _scale(y_pad, degT):
    return pl.pallas_call(
        _scale_body,
        grid=(R_PAD // BRP,),
        in_specs=[
            pl.BlockSpec((BRP, OUT_CH), lambda i: (i, 0)),
            pl.BlockSpec((BRP, 2), lambda i: (i, 0)),
        ],
        out_specs=[
            pl.BlockSpec((BRP, HALF), lambda i: (i, 0)),
            pl.BlockSpec((BRP, HALF), lambda i: (i, 0)),
        ],
        out_shape=[
            jax.ShapeDtypeStruct((R_PAD, HALF), jnp.float32),
            jax.ShapeDtypeStruct((R_PAD, HALF), jnp.float32),
        ],
    )(y_pad, degT)


def _final_body(acc0_ref, acc1_ref, hn0_ref, hn1_ref, degT_ref, bc_ref, out_ref):
    deg = degT_ref[:, 0:1] + degT_ref[:, 1:2] + 1.0
    dinv = lax.rsqrt(deg)
    lo = (acc0_ref[...] + hn0_ref[...]) * dinv + bc_ref[0:1, :]
    hi = (acc1_ref[...] + hn1_ref[...]) * dinv + bc_ref[1:2, :]
    out_ref[...] = jnp.concatenate([lo, hi], axis=1)


def _final(acc0, acc1, hn0, hn1, degT, bc2):
    return pl.pallas_call(
        _final_body,
        grid=(N // BR,),
        in_specs=[
            pl.BlockSpec((BR, HALF), lambda i: (i, 0)),
            pl.BlockSpec((BR, HALF), lambda i: (i, 0)),
            pl.BlockSpec((BR, HALF), lambda i: (i, 0)),
            pl.BlockSpec((BR, HALF), lambda i: (i, 0)),
            pl.BlockSpec((BR, 2), lambda i: (i, 0)),
            pl.BlockSpec((2, HALF), lambda i: (0, 0)),
        ],
        out_specs=pl.BlockSpec((BR, OUT_CH), lambda i: (i, 0)),
        out_shape=jax.ShapeDtypeStruct((N, OUT_CH), jnp.float32),
    )(acc0, acc1, hn0, hn1, degT, bc2)


def _sc_mesh():
    return plsc.VectorSubcoreMesh(core_axis_name="c", subcore_axis_name="s",
                                  num_cores=NC, num_subcores=NS)


def _deg_body(dst2_hbm, degp_hbm, dst2d_v, ones_v, dbuf_v, deg_sp):
    c = lax.axis_index("c")
    s = lax.axis_index("s")
    w = c * NS + s
    for i in range(CHUNK // L):
        ones_v[pl.ds(i * L, L)] = jnp.ones((L,), jnp.float32)

    def zfill(i, _):
        dbuf_v[pl.ds(i * L, L)] = jnp.zeros((L,), jnp.float32)
        return 0
    lax.fori_loop(0, RT // L, zfill, 0)
    pltpu.sync_copy(dbuf_v, deg_sp.at[pl.ds(s * RT, RT)])
    plsc.subcore_barrier()

    pltpu.sync_copy(dst2_hbm.at[pl.ds(w * CB, CB)], dst2d_v)

    def body(j, _):
        pltpu.sync_copy(ones_v, deg_sp.at[dst2d_v.at[j]], add=True)
        return 0
    lax.fori_loop(0, CB, body, 0)
    plsc.subcore_barrier()

    pltpu.sync_copy(deg_sp.at[pl.ds(s * RT, RT)], dbuf_v)
    pltpu.sync_copy(dbuf_v, degp_hbm.at[c].at[pl.ds(s * RT, RT)])


def _deg(dst2):
    f = pl.kernel(
        _deg_body,
        out_type=jax.ShapeDtypeStruct((NC, R_PAD), jnp.float32),
        mesh=_sc_mesh(),
        scratch_types=[
            pltpu.VMEM((CB, CHUNK), jnp.int32),
            pltpu.VMEM((CHUNK,), jnp.float32),
            pltpu.VMEM((RT,), jnp.float32),
            pltpu.VMEM_SHARED((R_PAD,), jnp.float32),
        ],
    )
    return f(dst2)


def _msg_body(src_hbm, dst2_hbm, hn0_hbm, hn1_hbm, acc0_hbm, acc1_hbm,
              src_v, dst2d_v, stg0, acc_sp):
    c = lax.axis_index("c")
    s = lax.axis_index("s")

    def zrow(i, _):
        for k in range(CHUNK // L):
            stg0[i, pl.ds(k * L, L)] = jnp.zeros((L,), jnp.float32)
        return 0
    lax.fori_loop(0, CHUNK, zrow, 0)
    for r in range(RT // CHUNK):
        pltpu.sync_copy(stg0, acc_sp.at[pl.ds(s * RT + r * CHUNK, CHUNK)])
    plsc.subcore_barrier()

    base = s * CD
    pltpu.sync_copy(src_hbm.at[pl.ds(base * CHUNK, CD * CHUNK)], src_v)
    pltpu.sync_copy(dst2_hbm.at[pl.ds(base, CD)], dst2d_v)

    def run(hn_hbm):
        def body(j, _):
            pltpu.sync_copy(hn_hbm.at[src_v.at[pl.ds(j * CHUNK, CHUNK)]], stg0)
            pltpu.sync_copy(stg0, acc_sp.at[dst2d_v.at[j]], add=True)
            return 0
        lax.fori_loop(0, CD, body, 0)

    @pl.when(c == 0)
    def _():
        run(hn0_hbm)

    @pl.when(c == 1)
    def _():
        run(hn1_hbm)

    plsc.subcore_barrier()

    def drain(out_hbm):
        for r in range(RT // CHUNK):
            pltpu.sync_copy(acc_sp.at[pl.ds(s * RT + r * CHUNK, CHUNK)], stg0)
            pltpu.sync_copy(stg0, out_hbm.at[pl.ds(s * RT + r * CHUNK, CHUNK)])

    @pl.when(c == 0)
    def _():
        drain(acc0_hbm)

    @pl.when(c == 1)
    def _():
        drain(acc1_hbm)


def _msg(srcp, dst2, hn0, hn1):
    f = pl.kernel(
        _msg_body,
        out_type=[
            jax.ShapeDtypeStruct((R_PAD, HALF), jnp.float32),
            jax.ShapeDtypeStruct((R_PAD, HALF), jnp.float32),
        ],
        mesh=_sc_mesh(),
        scratch_types=[
            pltpu.VMEM((CD * CHUNK,), jnp.int32),
            pltpu.VMEM((CD, CHUNK), jnp.int32),
            pltpu.VMEM((CHUNK, HALF), jnp.float32),
            pltpu.VMEM_SHARED((R_PAD, HALF), jnp.float32),
        ],
    )
    return f(srcp, dst2, hn0, hn1)


def kernel(x, edge_index, pos_embedding, Wp, Wc, bc):
    pad = E_PAD - E
    srcp = jnp.concatenate([edge_index[0], jnp.full((pad,), N, jnp.int32)])
    dstp = jnp.concatenate([edge_index[1], jnp.full((pad,), N, jnp.int32)])
    dst2 = dstp.reshape(C_T, CHUNK)

    y = _proj(x, pos_embedding, Wp, Wc)
    degp = _deg(dst2)
    degT = degp.T
    y_pad = jnp.pad(y, ((0, R_PAD - N), (0, 0)))
    hn0, hn1 = _scale(y_pad, degT)
    acc0, acc1 = _msg(srcp, dst2, hn0, hn1)
    bc2 = bc.reshape(2, HALF)
    return _final(acc0, acc1, hn0, hn1, degT, bc2)


# speedup vs baseline: 10.3415x; 10.3415x over previous
import functools

import jax
import jax.numpy as jnp
from jax import lax
from jax.experimental import pallas as pl
from jax.experimental.pallas import tpu as pltpu
from jax.experimental.pallas import tpu_sc as plsc

N = 10000
E = 320000
IN_CH = 256
OUT_CH = 256
POS = 8
HALF = OUT_CH // 2

NC = 2
NS = 16
L = 16

CHUNK = 128
R_PAD = 10240
C_T = 2560
E_PAD = C_T * CHUNK
CB = C_T // (NC * NS)
CD = C_T // NS
EG = 8
RT = R_PAD // NS

BR = 1000
BRP = 1024


def _proj_body(x_ref, pos_ref, wp_ref, wc_ref, y_ref):
    x = x_ref[...]
    pos = pos_ref[...]
    acc = jnp.zeros((BR, OUT_CH), jnp.float32)
    for p in range(POS):
        t = lax.dot_general(x, wp_ref[p * OUT_CH:(p + 1) * OUT_CH, :],
                            (((1,), (1,)), ((), ())),
                            preferred_element_type=jnp.float32)
        acc = acc + pos[:, p:p + 1] * t
    y_ref[...] = lax.dot_general(acc, wc_ref[...],
                                 (((1,), (1,)), ((), ())),
                                 preferred_element_type=jnp.float32)


def _proj(x, pos, Wp, Wc):
    return pl.pallas_call(
        _proj_body,
        grid=(N // BR,),
        in_specs=[
            pl.BlockSpec((BR, IN_CH), lambda i: (i, 0)),
            pl.BlockSpec((BR, POS), lambda i: (i, 0)),
            pl.BlockSpec((POS * OUT_CH, IN_CH), lambda i: (0, 0)),
            pl.BlockSpec((OUT_CH, OUT_CH), lambda i: (0, 0)),
        ],
        out_specs=pl.BlockSpec((BR, OUT_CH), lambda i: (i, 0)),
        out_shape=jax.ShapeDtypeStruct((N, OUT_CH), jnp.float32),
    )(x, pos, Wp, Wc)


def _scale_body(y_ref, degT_ref, hn0_ref, hn1_ref):
    deg = degT_ref[:, 0:1] + degT_ref[:, 1:2] + 1.0
    dinv = lax.rsqrt(deg)
    hn = y_ref[...] * dinv
    hn0_ref[...] = hn[:, :HALF]
    hn1_ref[...] = hn[:, HALF:]


def _scale(y_pad, degT):
    return pl.pallas_call(
        _scale_body,
        grid=(R_PAD // BRP,),
        in_specs=[
            pl.BlockSpec((BRP, OUT_CH), lambda i: (i, 0)),
            pl.BlockSpec((BRP, 2), lambda i: (i, 0)),
        ],
        out_specs=[
            pl.BlockSpec((BRP, HALF), lambda i: (i, 0)),
            pl.BlockSpec((BRP, HALF), lambda i: (i, 0)),
        ],
        out_shape=[
            jax.ShapeDtypeStruct((R_PAD, HALF), jnp.float32),
            jax.ShapeDtypeStruct((R_PAD, HALF), jnp.float32),
        ],
    )(y_pad, degT)


def _final_body(acc0_ref, acc1_ref, hn0_ref, hn1_ref, degT_ref, bc_ref, out_ref):
    deg = degT_ref[:, 0:1] + degT_ref[:, 1:2] + 1.0
    dinv = lax.rsqrt(deg)
    lo = (acc0_ref[...] + hn0_ref[...]) * dinv + bc_ref[0:1, :]
    hi = (acc1_ref[...] + hn1_ref[...]) * dinv + bc_ref[1:2, :]
    out_ref[...] = jnp.concatenate([lo, hi], axis=1)


def _final(acc0, acc1, hn0, hn1, degT, bc2):
    return pl.pallas_call(
        _final_body,
        grid=(N // BR,),
        in_specs=[
            pl.BlockSpec((BR, HALF), lambda i: (i, 0)),
            pl.BlockSpec((BR, HALF), lambda i: (i, 0)),
            pl.BlockSpec((BR, HALF), lambda i: (i, 0)),
            pl.BlockSpec((BR, HALF), lambda i: (i, 0)),
            pl.BlockSpec((BR, 2), lambda i: (i, 0)),
            pl.BlockSpec((2, HALF), lambda i: (0, 0)),
        ],
        out_specs=pl.BlockSpec((BR, OUT_CH), lambda i: (i, 0)),
        out_shape=jax.ShapeDtypeStruct((N, OUT_CH), jnp.float32),
    )(acc0, acc1, hn0, hn1, degT, bc2)


def _sc_mesh():
    return plsc.VectorSubcoreMesh(core_axis_name="c", subcore_axis_name="s",
                                  num_cores=NC, num_subcores=NS)


def _deg_body(dst2_hbm, degp_hbm, dst2d_v, ones_v, dbuf_v, deg_sp):
    c = lax.axis_index("c")
    s = lax.axis_index("s")
    w = c * NS + s
    for i in range(CHUNK // L):
        ones_v[pl.ds(i * L, L)] = jnp.ones((L,), jnp.float32)

    def zfill(i, _):
        dbuf_v[pl.ds(i * L, L)] = jnp.zeros((L,), jnp.float32)
        return 0
    lax.fori_loop(0, RT // L, zfill, 0)
    pltpu.sync_copy(dbuf_v, deg_sp.at[pl.ds(s * RT, RT)])
    plsc.subcore_barrier()

    pltpu.sync_copy(dst2_hbm.at[pl.ds(w * CB, CB)], dst2d_v)

    def body(j, _):
        pltpu.sync_copy(ones_v, deg_sp.at[dst2d_v.at[j]], add=True)
        return 0
    lax.fori_loop(0, CB, body, 0)
    plsc.subcore_barrier()

    pltpu.sync_copy(deg_sp.at[pl.ds(s * RT, RT)], dbuf_v)
    pltpu.sync_copy(dbuf_v, degp_hbm.at[pl.ds(c * R_PAD + s * RT, RT)])


def _deg(dst2):
    f = pl.kernel(
        _deg_body,
        out_type=jax.ShapeDtypeStruct((NC * R_PAD,), jnp.float32),
        mesh=_sc_mesh(),
        scratch_types=[
            pltpu.VMEM((CB, CHUNK), jnp.int32),
            pltpu.VMEM((CHUNK,), jnp.float32),
            pltpu.VMEM((RT,), jnp.float32),
            pltpu.VMEM_SHARED((R_PAD,), jnp.float32),
        ],
    )
    return f(dst2)


def _msg_body(src_hbm, dst2_hbm, hn0_hbm, hn1_hbm, acc0_hbm, acc1_hbm,
              src_v, dst2d_v, stg0, acc_sp):
    c = lax.axis_index("c")
    s = lax.axis_index("s")

    def zrow(i, _):
        for k in range(CHUNK // L):
            stg0[i, pl.ds(k * L, L)] = jnp.zeros((L,), jnp.float32)
        return 0
    lax.fori_loop(0, CHUNK, zrow, 0)
    for r in range(RT // CHUNK):
        pltpu.sync_copy(stg0, acc_sp.at[pl.ds(s * RT + r * CHUNK, CHUNK)])
    plsc.subcore_barrier()

    def run(hn_hbm):
        def body(g, _):
            base = pl.multiple_of(s * CD + g * EG, 8)
            pltpu.sync_copy(src_hbm.at[pl.ds(base * CHUNK, EG * CHUNK)], src_v)
            pltpu.sync_copy(dst2_hbm.at[pl.ds(base, EG)], dst2d_v)
            for j in range(EG):
                pltpu.sync_copy(hn_hbm.at[src_v.at[pl.ds(j * CHUNK, CHUNK)]], stg0)
                pltpu.sync_copy(stg0, acc_sp.at[dst2d_v.at[j]], add=True)
            return 0
        lax.fori_loop(0, CD // EG, body, 0)

    @pl.when(c == 0)
    def _():
        run(hn0_hbm)

    @pl.when(c == 1)
    def _():
        run(hn1_hbm)

    plsc.subcore_barrier()

    def drain(out_hbm):
        for r in range(RT // CHUNK):
            pltpu.sync_copy(acc_sp.at[pl.ds(s * RT + r * CHUNK, CHUNK)], stg0)
            pltpu.sync_copy(stg0, out_hbm.at[pl.ds(s * RT + r * CHUNK, CHUNK)])

    @pl.when(c == 0)
    def _():
        drain(acc0_hbm)

    @pl.when(c == 1)
    def _():
        drain(acc1_hbm)


def _msg(srcp, dst2, hn0, hn1):
    f = pl.kernel(
        _msg_body,
        out_type=[
            jax.ShapeDtypeStruct((R_PAD, HALF), jnp.float32),
            jax.ShapeDtypeStruct((R_PAD, HALF), jnp.float32),
        ],
        mesh=_sc_mesh(),
        scratch_types=[
            pltpu.VMEM((EG * CHUNK,), jnp.int32),
            pltpu.VMEM((EG, CHUNK), jnp.int32),
            pltpu.VMEM((CHUNK, HALF), jnp.float32),
            pltpu.VMEM_SHARED((R_PAD, HALF), jnp.float32),
        ],
    )
    return f(srcp, dst2, hn0, hn1)


def kernel(x, edge_index, pos_embedding, Wp, Wc, bc):
    pad = E_PAD - E
    srcp = jnp.concatenate([edge_index[0], jnp.full((pad,), N, jnp.int32)])
    dstp = jnp.concatenate([edge_index[1], jnp.full((pad,), N, jnp.int32)])
    dst2 = dstp.reshape(C_T, CHUNK)

    y = _proj(x, pos_embedding, Wp, Wc)
    degp = _deg(dst2)
    degT = degp.reshape(NC, R_PAD).T
    y_pad = jnp.pad(y, ((0, R_PAD - N), (0, 0)))
    hn0, hn1 = _scale(y_pad, degT)
    acc0, acc1 = _msg(srcp, dst2, hn0, hn1)
    bc2 = bc.reshape(2, HALF)
    return _final(acc0, acc1, hn0, hn1, degT, bc2)


# speedup vs baseline: 23.8427x; 2.3055x over previous
import functools

import jax
import jax.numpy as jnp
from jax import lax
from jax.experimental import pallas as pl
from jax.experimental.pallas import tpu as pltpu
from jax.experimental.pallas import tpu_sc as plsc

N = 10000
E = 320000
IN_CH = 256
OUT_CH = 256
POS = 8
HALF = OUT_CH // 2

NC = 2
NS = 16
L = 16

CHUNK = 128
R_PAD = 10240
C_T = 2560
E_PAD = C_T * CHUNK
CB = C_T // (NC * NS)
CD = C_T // NS
EG = 32
NBUF = 2
RT = R_PAD // NS

BR = 1000
BRP = 1024


def _proj_body(x_ref, pos_ref, wp_ref, wc_ref, y_ref):
    x = x_ref[...]
    pos = pos_ref[...]
    acc = jnp.zeros((BR, OUT_CH), jnp.float32)
    for p in range(POS):
        t = lax.dot_general(x, wp_ref[p * OUT_CH:(p + 1) * OUT_CH, :],
                            (((1,), (1,)), ((), ())),
                            preferred_element_type=jnp.float32)
        acc = acc + pos[:, p:p + 1] * t
    y_ref[...] = lax.dot_general(acc, wc_ref[...],
                                 (((1,), (1,)), ((), ())),
                                 preferred_element_type=jnp.float32)


def _proj(x, pos, Wp, Wc):
    return pl.pallas_call(
        _proj_body,
        grid=(N // BR,),
        in_specs=[
            pl.BlockSpec((BR, IN_CH), lambda i: (i, 0)),
            pl.BlockSpec((BR, POS), lambda i: (i, 0)),
            pl.BlockSpec((POS * OUT_CH, IN_CH), lambda i: (0, 0)),
            pl.BlockSpec((OUT_CH, OUT_CH), lambda i: (0, 0)),
        ],
        out_specs=pl.BlockSpec((BR, OUT_CH), lambda i: (i, 0)),
        out_shape=jax.ShapeDtypeStruct((N, OUT_CH), jnp.float32),
    )(x, pos, Wp, Wc)


def _scale_body(y_ref, degT_ref, hn0_ref, hn1_ref):
    deg = degT_ref[:, 0:1] + degT_ref[:, 1:2] + 1.0
    dinv = lax.rsqrt(deg)
    hn = y_ref[...] * dinv
    hn0_ref[...] = hn[:, :HALF]
    hn1_ref[...] = hn[:, HALF:]


def _scale(y_pad, degT):
    return pl.pallas_call(
        _scale_body,
        grid=(R_PAD // BRP,),
        in_specs=[
            pl.BlockSpec((BRP, OUT_CH), lambda i: (i, 0)),
            pl.BlockSpec((BRP, 2), lambda i: (i, 0)),
        ],
        out_specs=[
            pl.BlockSpec((BRP, HALF), lambda i: (i, 0)),
            pl.BlockSpec((BRP, HALF), lambda i: (i, 0)),
        ],
        out_shape=[
            jax.ShapeDtypeStruct((R_PAD, HALF), jnp.float32),
            jax.ShapeDtypeStruct((R_PAD, HALF), jnp.float32),
        ],
    )(y_pad, degT)


def _final_body(acc0_ref, acc1_ref, hn0_ref, hn1_ref, degT_ref, bc_ref, out_ref):
    deg = degT_ref[:, 0:1] + degT_ref[:, 1:2] + 1.0
    dinv = lax.rsqrt(deg)
    lo = (acc0_ref[...] + hn0_ref[...]) * dinv + bc_ref[0:1, :]
    hi = (acc1_ref[...] + hn1_ref[...]) * dinv + bc_ref[1:2, :]
    out_ref[...] = jnp.concatenate([lo, hi], axis=1)


def _final(acc0, acc1, hn0, hn1, degT, bc2):
    return pl.pallas_call(
        _final_body,
        grid=(N // BR,),
        in_specs=[
            pl.BlockSpec((BR, HALF), lambda i: (i, 0)),
            pl.BlockSpec((BR, HALF), lambda i: (i, 0)),
            pl.BlockSpec((BR, HALF), lambda i: (i, 0)),
            pl.BlockSpec((BR, HALF), lambda i: (i, 0)),
            pl.BlockSpec((BR, 2), lambda i: (i, 0)),
            pl.BlockSpec((2, HALF), lambda i: (0, 0)),
        ],
        out_specs=pl.BlockSpec((BR, OUT_CH), lambda i: (i, 0)),
        out_shape=jax.ShapeDtypeStruct((N, OUT_CH), jnp.float32),
    )(acc0, acc1, hn0, hn1, degT, bc2)


def _sc_mesh():
    return plsc.VectorSubcoreMesh(core_axis_name="c", subcore_axis_name="s",
                                  num_cores=NC, num_subcores=NS)


def _deg_body(dst2_hbm, degp_hbm, dst2d_v, ones_v, dbuf_v, deg_sp):
    c = lax.axis_index("c")
    s = lax.axis_index("s")
    w = c * NS + s
    for i in range(CHUNK // L):
        ones_v[pl.ds(i * L, L)] = jnp.ones((L,), jnp.float32)

    def zfill(i, _):
        dbuf_v[pl.ds(i * L, L)] = jnp.zeros((L,), jnp.float32)
        return 0
    lax.fori_loop(0, RT // L, zfill, 0)
    pltpu.sync_copy(dbuf_v, deg_sp.at[pl.ds(s * RT, RT)])
    plsc.subcore_barrier()

    pltpu.sync_copy(dst2_hbm.at[pl.ds(w * CB, CB)], dst2d_v)

    def body(j, _):
        pltpu.sync_copy(ones_v, deg_sp.at[dst2d_v.at[j]], add=True)
        return 0
    lax.fori_loop(0, CB, body, 0)
    plsc.subcore_barrier()

    pltpu.sync_copy(deg_sp.at[pl.ds(s * RT, RT)], dbuf_v)
    pltpu.sync_copy(dbuf_v, degp_hbm.at[pl.ds(c * R_PAD + s * RT, RT)])


def _deg(dst2):
    f = pl.kernel(
        _deg_body,
        out_type=jax.ShapeDtypeStruct((NC * R_PAD,), jnp.float32),
        mesh=_sc_mesh(),
        scratch_types=[
            pltpu.VMEM((CB, CHUNK), jnp.int32),
            pltpu.VMEM((CHUNK,), jnp.float32),
            pltpu.VMEM((RT,), jnp.float32),
            pltpu.VMEM_SHARED((R_PAD,), jnp.float32),
        ],
    )
    return f(dst2)


def _msg_body(src_hbm, dst2_hbm, hn0_hbm, hn1_hbm, acc0_hbm, acc1_hbm,
              src_v, dst2d_v, stg0, stg1, acc_sp, gs0, gs1, ss0, ss1):
    c = lax.axis_index("c")
    s = lax.axis_index("s")
    stg = (stg0, stg1)
    gsem = (gs0, gs1)
    ssem = (ss0, ss1)

    def zrow(i, _):
        for k in range(CHUNK // L):
            stg0[i, pl.ds(k * L, L)] = jnp.zeros((L,), jnp.float32)
        return 0
    lax.fori_loop(0, CHUNK, zrow, 0)
    for r in range(RT // CHUNK):
        pltpu.sync_copy(stg0, acc_sp.at[pl.ds(s * RT + r * CHUNK, CHUNK)])
    plsc.subcore_barrier()

    def run(hn_hbm):
        def gwait(slot, j):
            pltpu.make_async_copy(
                hn_hbm.at[src_v.at[pl.ds(j * CHUNK, CHUNK)]],
                stg[slot], gsem[slot]).wait()

        def swait(slot, j):
            pltpu.make_async_copy(stg[slot], acc_sp.at[dst2d_v.at[j]],
                                  ssem[slot]).wait()

        def group(g, _):
            base = pl.multiple_of(s * CD + g * EG, 8)
            pltpu.sync_copy(src_hbm.at[pl.ds(base * CHUNK, EG * CHUNK)], src_v)
            pltpu.sync_copy(dst2_hbm.at[pl.ds(base, EG)], dst2d_v)
            for slot in range(NBUF):
                pltpu.async_copy(
                    hn_hbm.at[src_v.at[pl.ds(slot * CHUNK, CHUNK)]],
                    stg[slot], gsem[slot])

            def pair(p, _):
                for slot in range(NBUF):
                    j = p * NBUF + slot
                    gwait(slot, j)
                    pltpu.async_copy(stg[slot], acc_sp.at[dst2d_v.at[j]],
                                     ssem[slot], add=True)
                for slot in range(NBUF):
                    j = p * NBUF + slot

                    @pl.when(j + NBUF < EG)
                    def _():
                        swait(slot, j)
                        pltpu.async_copy(
                            hn_hbm.at[src_v.at[pl.ds((j + NBUF) * CHUNK, CHUNK)]],
                            stg[slot], gsem[slot])
                return 0
            lax.fori_loop(0, EG // NBUF, pair, 0)
            for slot in range(NBUF):
                swait(slot, EG - NBUF + slot)
            return 0
        lax.fori_loop(0, CD // EG, group, 0)

    @pl.when(c == 0)
    def _():
        run(hn0_hbm)

    @pl.when(c == 1)
    def _():
        run(hn1_hbm)

    plsc.subcore_barrier()

    def drain(out_hbm):
        for r in range(RT // CHUNK):
            pltpu.sync_copy(acc_sp.at[pl.ds(s * RT + r * CHUNK, CHUNK)], stg0)
            pltpu.sync_copy(stg0, out_hbm.at[pl.ds(s * RT + r * CHUNK, CHUNK)])

    @pl.when(c == 0)
    def _():
        drain(acc0_hbm)

    @pl.when(c == 1)
    def _():
        drain(acc1_hbm)


def _msg(srcp, dst2, hn0, hn1):
    f = pl.kernel(
        _msg_body,
        out_type=[
            jax.ShapeDtypeStruct((R_PAD, HALF), jnp.float32),
            jax.ShapeDtypeStruct((R_PAD, HALF), jnp.float32),
        ],
        mesh=_sc_mesh(),
        scratch_types=[
            pltpu.VMEM((EG * CHUNK,), jnp.int32),
            pltpu.VMEM((EG, CHUNK), jnp.int32),
            pltpu.VMEM((CHUNK, HALF), jnp.float32),
            pltpu.VMEM((CHUNK, HALF), jnp.float32),
            pltpu.VMEM_SHARED((R_PAD, HALF), jnp.float32),
            pltpu.SemaphoreType.DMA,
            pltpu.SemaphoreType.DMA,
            pltpu.SemaphoreType.DMA,
            pltpu.SemaphoreType.DMA,
        ],
    )
    return f(srcp, dst2, hn0, hn1)


def kernel(x, edge_index, pos_embedding, Wp, Wc, bc):
    pad = E_PAD - E
    dummy = N + jnp.arange(pad, dtype=jnp.int32) % (R_PAD - N)
    srcp = jnp.concatenate([edge_index[0], dummy])
    dstp = jnp.concatenate([edge_index[1], dummy])
    dst2 = dstp.reshape(C_T, CHUNK)

    y = _proj(x, pos_embedding, Wp, Wc)
    degp = _deg(dst2)
    degT = degp.reshape(NC, R_PAD).T
    y_pad = jnp.pad(y, ((0, R_PAD - N), (0, 0)))
    hn0, hn1 = _scale(y_pad, degT)
    acc0, acc1 = _msg(srcp, dst2, hn0, hn1)
    bc2 = bc.reshape(2, HALF)
    return _final(acc0, acc1, hn0, hn1, degT, bc2)


# speedup vs baseline: 25.9116x; 1.0868x over previous
import functools

import jax
import jax.numpy as jnp
from jax import lax
from jax.experimental import pallas as pl
from jax.experimental.pallas import tpu as pltpu
from jax.experimental.pallas import tpu_sc as plsc

N = 10000
E = 320000
IN_CH = 256
OUT_CH = 256
POS = 8
HALF = OUT_CH // 2

NC = 2
NS = 16
L = 16

CHUNK = 128
R_PAD = 10240
C_T = 2560
E_PAD = C_T * CHUNK
CB = C_T // (NC * NS)
CD = C_T // NS
EG = 32
NBUF = 2
RT = R_PAD // NS

BR = 1000
BRP = 1024


def _proj_body(x_ref, pos_ref, wp_ref, wc_ref, y_ref):
    x = x_ref[...]
    pos = pos_ref[...]
    acc = jnp.zeros((BR, OUT_CH), jnp.float32)
    for p in range(POS):
        t = lax.dot_general(x, wp_ref[p * OUT_CH:(p + 1) * OUT_CH, :],
                            (((1,), (1,)), ((), ())),
                            preferred_element_type=jnp.float32)
        acc = acc + pos[:, p:p + 1] * t
    y_ref[...] = lax.dot_general(acc, wc_ref[...],
                                 (((1,), (1,)), ((), ())),
                                 preferred_element_type=jnp.float32)


def _proj(x, pos, Wp, Wc):
    return pl.pallas_call(
        _proj_body,
        grid=(N // BR,),
        in_specs=[
            pl.BlockSpec((BR, IN_CH), lambda i: (i, 0)),
            pl.BlockSpec((BR, POS), lambda i: (i, 0)),
            pl.BlockSpec((POS * OUT_CH, IN_CH), lambda i: (0, 0)),
            pl.BlockSpec((OUT_CH, OUT_CH), lambda i: (0, 0)),
        ],
        out_specs=pl.BlockSpec((BR, OUT_CH), lambda i: (i, 0)),
        out_shape=jax.ShapeDtypeStruct((N, OUT_CH), jnp.float32),
    )(x, pos, Wp, Wc)


def _scale_body(y_ref, degT_ref, hn0_ref, hn1_ref):
    deg = degT_ref[:, 0:1] + degT_ref[:, 1:2] + 1.0
    dinv = lax.rsqrt(deg)
    hn = y_ref[...] * dinv
    hn0_ref[...] = hn[:, :HALF]
    hn1_ref[...] = hn[:, HALF:]


def _scale(y_pad, degT):
    return pl.pallas_call(
        _scale_body,
        grid=(R_PAD // BRP,),
        in_specs=[
            pl.BlockSpec((BRP, OUT_CH), lambda i: (i, 0)),
            pl.BlockSpec((BRP, 2), lambda i: (i, 0)),
        ],
        out_specs=[
            pl.BlockSpec((BRP, HALF), lambda i: (i, 0)),
            pl.BlockSpec((BRP, HALF), lambda i: (i, 0)),
        ],
        out_shape=[
            jax.ShapeDtypeStruct((R_PAD, HALF), jnp.float32),
            jax.ShapeDtypeStruct((R_PAD, HALF), jnp.float32),
        ],
    )(y_pad, degT)


def _final_body(acc0_ref, acc1_ref, hn0_ref, hn1_ref, degT_ref, bc_ref, out_ref):
    deg = degT_ref[:, 0:1] + degT_ref[:, 1:2] + 1.0
    dinv = lax.rsqrt(deg)
    lo = (acc0_ref[...] + hn0_ref[...]) * dinv + bc_ref[0:1, :]
    hi = (acc1_ref[...] + hn1_ref[...]) * dinv + bc_ref[1:2, :]
    out_ref[...] = jnp.concatenate([lo, hi], axis=1)


def _final(acc0, acc1, hn0, hn1, degT, bc2):
    return pl.pallas_call(
        _final_body,
        grid=(N // BR,),
        in_specs=[
            pl.BlockSpec((BR, HALF), lambda i: (i, 0)),
            pl.BlockSpec((BR, HALF), lambda i: (i, 0)),
            pl.BlockSpec((BR, HALF), lambda i: (i, 0)),
            pl.BlockSpec((BR, HALF), lambda i: (i, 0)),
            pl.BlockSpec((BR, 2), lambda i: (i, 0)),
            pl.BlockSpec((2, HALF), lambda i: (0, 0)),
        ],
        out_specs=pl.BlockSpec((BR, OUT_CH), lambda i: (i, 0)),
        out_shape=jax.ShapeDtypeStruct((N, OUT_CH), jnp.float32),
    )(acc0, acc1, hn0, hn1, degT, bc2)


def _sc_mesh():
    return plsc.VectorSubcoreMesh(core_axis_name="c", subcore_axis_name="s",
                                  num_cores=NC, num_subcores=NS)


def _deg_body(dst2_hbm, degp_hbm, dst2d_v, ones_v, dbuf_v, deg_sp):
    c = lax.axis_index("c")
    s = lax.axis_index("s")
    w = c * NS + s
    for i in range(CHUNK // L):
        ones_v[pl.ds(i * L, L)] = jnp.ones((L,), jnp.float32)

    def zfill(i, _):
        dbuf_v[pl.ds(i * L, L)] = jnp.zeros((L,), jnp.float32)
        return 0
    lax.fori_loop(0, RT // L, zfill, 0)
    pltpu.sync_copy(dbuf_v, deg_sp.at[pl.ds(s * RT, RT)])
    plsc.subcore_barrier()

    pltpu.sync_copy(dst2_hbm.at[pl.ds(w * CB, CB)], dst2d_v)

    def body(j, _):
        pltpu.sync_copy(ones_v, deg_sp.at[dst2d_v.at[j]], add=True)
        return 0
    lax.fori_loop(0, CB, body, 0)
    plsc.subcore_barrier()

    pltpu.sync_copy(deg_sp.at[pl.ds(s * RT, RT)], dbuf_v)
    pltpu.sync_copy(dbuf_v, degp_hbm.at[pl.ds(c * R_PAD + s * RT, RT)])


def _deg(dst2):
    f = pl.kernel(
        _deg_body,
        out_type=jax.ShapeDtypeStruct((NC * R_PAD,), jnp.float32),
        mesh=_sc_mesh(),
        scratch_types=[
            pltpu.VMEM((CB, CHUNK), jnp.int32),
            pltpu.VMEM((CHUNK,), jnp.float32),
            pltpu.VMEM((RT,), jnp.float32),
            pltpu.VMEM_SHARED((R_PAD,), jnp.float32),
        ],
    )
    return f(dst2)


def _msg_body(src_hbm, dst2_hbm, hn0_hbm, hn1_hbm, acc0_hbm, acc1_hbm,
              src_v, dst2d_v, stg0, stg1, acc_sp, gs0, gs1, ss0, ss1):
    c = lax.axis_index("c")
    s = lax.axis_index("s")
    stg = (stg0, stg1)
    gsem = (gs0, gs1)
    ssem = (ss0, ss1)

    def zrow(i, _):
        for k in range(CHUNK // L):
            stg0[i, pl.ds(k * L, L)] = jnp.zeros((L,), jnp.float32)
        return 0
    lax.fori_loop(0, CHUNK, zrow, 0)
    for r in range(RT // CHUNK):
        pltpu.sync_copy(stg0, acc_sp.at[pl.ds(s * RT + r * CHUNK, CHUNK)])
    plsc.subcore_barrier()

    def run(hn_hbm):
        def gwait(slot, j):
            pltpu.make_async_copy(
                hn_hbm.at[src_v.at[pl.ds(j * CHUNK, CHUNK)]],
                stg[slot], gsem[slot]).wait()

        def swait(slot, j):
            pltpu.make_async_copy(stg[slot], acc_sp.at[dst2d_v.at[j]],
                                  ssem[slot]).wait()

        def gissue(slot, j):
            pltpu.async_copy(hn_hbm.at[src_v.at[pl.ds(j * CHUNK, CHUNK)]],
                             stg[slot], gsem[slot])

        def group(g, _):
            base = pl.multiple_of(s * CD + g * EG, 8)
            pltpu.sync_copy(src_hbm.at[pl.ds(base * CHUNK, EG * CHUNK)], src_v)
            pltpu.sync_copy(dst2_hbm.at[pl.ds(base, EG)], dst2d_v)
            gissue(0, 0)

            def pair(p, _):
                for slot in range(NBUF):
                    j = p * NBUF + slot
                    oslot = 1 - slot
                    gwait(slot, j)
                    pltpu.async_copy(stg[slot], acc_sp.at[dst2d_v.at[j]],
                                     ssem[slot], add=True)

                    @pl.when(jnp.logical_and(j > 0, j + 1 < EG))
                    def _():
                        swait(oslot, j - 1)
                        gissue(oslot, j + 1)

                    @pl.when(j == 0)
                    def _():
                        gissue(oslot, 1)
                return 0
            lax.fori_loop(0, EG // NBUF, pair, 0)
            swait((EG - 2) % NBUF, EG - 2)
            swait((EG - 1) % NBUF, EG - 1)
            return 0
        lax.fori_loop(0, CD // EG, group, 0)

    @pl.when(c == 0)
    def _():
        run(hn0_hbm)

    @pl.when(c == 1)
    def _():
        run(hn1_hbm)

    plsc.subcore_barrier()

    def drain(out_hbm):
        for r in range(RT // CHUNK):
            pltpu.sync_copy(acc_sp.at[pl.ds(s * RT + r * CHUNK, CHUNK)], stg0)
            pltpu.sync_copy(stg0, out_hbm.at[pl.ds(s * RT + r * CHUNK, CHUNK)])

    @pl.when(c == 0)
    def _():
        drain(acc0_hbm)

    @pl.when(c == 1)
    def _():
        drain(acc1_hbm)


def _msg(srcp, dst2, hn0, hn1):
    f = pl.kernel(
        _msg_body,
        out_type=[
            jax.ShapeDtypeStruct((R_PAD, HALF), jnp.float32),
            jax.ShapeDtypeStruct((R_PAD, HALF), jnp.float32),
        ],
        mesh=_sc_mesh(),
        scratch_types=[
            pltpu.VMEM((EG * CHUNK,), jnp.int32),
            pltpu.VMEM((EG, CHUNK), jnp.int32),
            pltpu.VMEM((CHUNK, HALF), jnp.float32),
            pltpu.VMEM((CHUNK, HALF), jnp.float32),
            pltpu.VMEM_SHARED((R_PAD, HALF), jnp.float32),
            pltpu.SemaphoreType.DMA,
            pltpu.SemaphoreType.DMA,
            pltpu.SemaphoreType.DMA,
            pltpu.SemaphoreType.DMA,
        ],
    )
    return f(srcp, dst2, hn0, hn1)


def kernel(x, edge_index, pos_embedding, Wp, Wc, bc):
    pad = E_PAD - E
    dummy = N + jnp.arange(pad, dtype=jnp.int32) % (R_PAD - N)
    srcp = jnp.concatenate([edge_index[0], dummy])
    dstp = jnp.concatenate([edge_index[1], dummy])
    dst2 = dstp.reshape(C_T, CHUNK)

    y = _proj(x, pos_embedding, Wp, Wc)
    degp = _deg(dst2)
    degT = degp.reshape(NC, R_PAD).T
    y_pad = jnp.pad(y, ((0, R_PAD - N), (0, 0)))
    hn0, hn1 = _scale(y_pad, degT)
    acc0, acc1 = _msg(srcp, dst2, hn0, hn1)
    bc2 = bc.reshape(2, HALF)
    return _final(acc0, acc1, hn0, hn1, degT, bc2)


# speedup vs baseline: 26.6310x; 1.0278x over previous
import functools

import jax
import jax.numpy as jnp
from jax import lax
from jax.experimental import pallas as pl
from jax.experimental.pallas import tpu as pltpu
from jax.experimental.pallas import tpu_sc as plsc

N = 10000
E = 320000
IN_CH = 256
OUT_CH = 256
POS = 8
HALF = OUT_CH // 2

NC = 2
NS = 16
L = 16

CHUNK = 128
R_PAD = 10240
C_T = 2560
E_PAD = C_T * CHUNK
CB = C_T // (NC * NS)
CD = C_T // NS
EG = 16
NG = CD // EG
NBUF = 2
RT = R_PAD // NS

BR = 1000
BRP = 1024


def _proj_body(x_ref, pos_ref, wp_ref, wc_ref, y_ref):
    x = x_ref[...]
    pos = pos_ref[...]
    acc = jnp.zeros((BR, OUT_CH), jnp.float32)
    for p in range(POS):
        t = lax.dot_general(x, wp_ref[p * OUT_CH:(p + 1) * OUT_CH, :],
                            (((1,), (1,)), ((), ())),
                            preferred_element_type=jnp.float32)
        acc = acc + pos[:, p:p + 1] * t
    y_ref[...] = lax.dot_general(acc, wc_ref[...],
                                 (((1,), (1,)), ((), ())),
                                 preferred_element_type=jnp.float32)


def _proj(x, pos, Wp, Wc):
    return pl.pallas_call(
        _proj_body,
        grid=(N // BR,),
        in_specs=[
            pl.BlockSpec((BR, IN_CH), lambda i: (i, 0)),
            pl.BlockSpec((BR, POS), lambda i: (i, 0)),
            pl.BlockSpec((POS * OUT_CH, IN_CH), lambda i: (0, 0)),
            pl.BlockSpec((OUT_CH, OUT_CH), lambda i: (0, 0)),
        ],
        out_specs=pl.BlockSpec((BR, OUT_CH), lambda i: (i, 0)),
        out_shape=jax.ShapeDtypeStruct((N, OUT_CH), jnp.float32),
    )(x, pos, Wp, Wc)


def _scale_body(y_ref, degT_ref, hn0_ref, hn1_ref):
    deg = degT_ref[:, 0:1] + degT_ref[:, 1:2] + 1.0
    dinv = lax.rsqrt(deg)
    hn = y_ref[...] * dinv
    hn0_ref[...] = hn[:, :HALF]
    hn1_ref[...] = hn[:, HALF:]


def _scale(y, degT):
    return pl.pallas_call(
        _scale_body,
        grid=(R_PAD // BRP,),
        in_specs=[
            pl.BlockSpec((BRP, OUT_CH), lambda i: (i, 0)),
            pl.BlockSpec((BRP, 2), lambda i: (i, 0)),
        ],
        out_specs=[
            pl.BlockSpec((BRP, HALF), lambda i: (i, 0)),
            pl.BlockSpec((BRP, HALF), lambda i: (i, 0)),
        ],
        out_shape=[
            jax.ShapeDtypeStruct((R_PAD, HALF), jnp.float32),
            jax.ShapeDtypeStruct((R_PAD, HALF), jnp.float32),
        ],
    )(y, degT)


def _final_body(acc0_ref, acc1_ref, hn0_ref, hn1_ref, degT_ref, bc_ref, out_ref):
    deg = degT_ref[:, 0:1] + degT_ref[:, 1:2] + 1.0
    dinv = lax.rsqrt(deg)
    lo = (acc0_ref[...] + hn0_ref[...]) * dinv + bc_ref[0:1, :]
    hi = (acc1_ref[...] + hn1_ref[...]) * dinv + bc_ref[1:2, :]
    out_ref[...] = jnp.concatenate([lo, hi], axis=1)


def _final(acc0, acc1, hn0, hn1, degT, bc2):
    return pl.pallas_call(
        _final_body,
        grid=(N // BR,),
        in_specs=[
            pl.BlockSpec((BR, HALF), lambda i: (i, 0)),
            pl.BlockSpec((BR, HALF), lambda i: (i, 0)),
            pl.BlockSpec((BR, HALF), lambda i: (i, 0)),
            pl.BlockSpec((BR, HALF), lambda i: (i, 0)),
            pl.BlockSpec((BR, 2), lambda i: (i, 0)),
            pl.BlockSpec((2, HALF), lambda i: (0, 0)),
        ],
        out_specs=pl.BlockSpec((BR, OUT_CH), lambda i: (i, 0)),
        out_shape=jax.ShapeDtypeStruct((N, OUT_CH), jnp.float32),
    )(acc0, acc1, hn0, hn1, degT, bc2)


def _sc_mesh():
    return plsc.VectorSubcoreMesh(core_axis_name="c", subcore_axis_name="s",
                                  num_cores=NC, num_subcores=NS)


def _deg_body(dst2_hbm, degp_hbm, dst2d_v, ones_v, dbuf_v, deg_sp, dsem):
    c = lax.axis_index("c")
    s = lax.axis_index("s")
    w = c * NS + s
    for i in range(CHUNK // L):
        ones_v[pl.ds(i * L, L)] = jnp.ones((L,), jnp.float32)

    def zfill(i, _):
        dbuf_v[pl.ds(i * L, L)] = jnp.zeros((L,), jnp.float32)
        return 0
    lax.fori_loop(0, RT // L, zfill, 0)
    pltpu.sync_copy(dbuf_v, deg_sp.at[pl.ds(s * RT, RT)])
    plsc.subcore_barrier()

    pltpu.sync_copy(dst2_hbm.at[pl.ds(w * CB, CB)], dst2d_v)

    DB = 8

    def body(q, _):
        for k in range(DB):
            pltpu.async_copy(ones_v, deg_sp.at[dst2d_v.at[q * DB + k]],
                             dsem, add=True)
        for k in range(DB):
            pltpu.make_async_copy(ones_v, deg_sp.at[dst2d_v.at[q * DB + k]],
                                  dsem).wait()
        return 0
    lax.fori_loop(0, CB // DB, body, 0)
    plsc.subcore_barrier()

    pltpu.sync_copy(deg_sp.at[pl.ds(s * RT, RT)], dbuf_v)
    pltpu.sync_copy(dbuf_v, degp_hbm.at[pl.ds(c * R_PAD + s * RT, RT)])


def _deg(dst2):
    f = pl.kernel(
        _deg_body,
        out_type=jax.ShapeDtypeStruct((NC * R_PAD,), jnp.float32),
        mesh=_sc_mesh(),
        scratch_types=[
            pltpu.VMEM((CB, CHUNK), jnp.int32),
            pltpu.VMEM((CHUNK,), jnp.float32),
            pltpu.VMEM((RT,), jnp.float32),
            pltpu.VMEM_SHARED((R_PAD,), jnp.float32),
            pltpu.SemaphoreType.DMA,
        ],
    )
    return f(dst2)


def _msg_body(src_hbm, dst2_hbm, hn0_hbm, hn1_hbm, acc0_hbm, acc1_hbm,
              src_v, dst2d_v, stg0, stg1, acc_sp, gs0, gs1, ss0, ss1, isem):
    c = lax.axis_index("c")
    s = lax.axis_index("s")
    stg = (stg0, stg1)
    gsem = (gs0, gs1)
    ssem = (ss0, ss1)

    def zrow(i, _):
        for k in range(CHUNK // L):
            stg0[i, pl.ds(k * L, L)] = jnp.zeros((L,), jnp.float32)
        return 0
    lax.fori_loop(0, CHUNK, zrow, 0)
    for r in range(RT // CHUNK):
        pltpu.sync_copy(stg0, acc_sp.at[pl.ds(s * RT + r * CHUNK, CHUNK)])
    plsc.subcore_barrier()

    def idx_fetch(g, b):
        base = pl.multiple_of(s * CD + g * EG, 8)
        pltpu.async_copy(src_hbm.at[pl.ds(base * CHUNK, EG * CHUNK)],
                         src_v.at[b], isem)
        pltpu.async_copy(dst2_hbm.at[pl.ds(base, EG)], dst2d_v.at[b], isem)

    def idx_wait(b):
        pltpu.make_async_copy(src_hbm.at[pl.ds(0, EG * CHUNK)],
                              src_v.at[b], isem).wait()
        pltpu.make_async_copy(dst2_hbm.at[pl.ds(0, EG)],
                              dst2d_v.at[b], isem).wait()

    def run(hn_hbm):
        def gwait(slot, b, j):
            pltpu.make_async_copy(
                hn_hbm.at[src_v.at[b].at[pl.ds(j * CHUNK, CHUNK)]],
                stg[slot], gsem[slot]).wait()

        def swait(slot, b, j):
            pltpu.make_async_copy(stg[slot], acc_sp.at[dst2d_v.at[b].at[j]],
                                  ssem[slot]).wait()

        def gissue(slot, b, j):
            pltpu.async_copy(
                hn_hbm.at[src_v.at[b].at[pl.ds(j * CHUNK, CHUNK)]],
                stg[slot], gsem[slot])

        def ring(g, b):
            gissue(0, b, 0)

            def pair(p, _):
                for slot in range(NBUF):
                    j = p * NBUF + slot
                    oslot = 1 - slot
                    gwait(slot, b, j)
                    pltpu.async_copy(stg[slot],
                                     acc_sp.at[dst2d_v.at[b].at[j]],
                                     ssem[slot], add=True)

                    @pl.when(jnp.logical_and(j > 0, j + 1 < EG))
                    def _():
                        swait(oslot, b, j - 1)
                        gissue(oslot, b, j + 1)

                    @pl.when(j == 0)
                    def _():
                        gissue(oslot, b, 1)
                return 0
            lax.fori_loop(0, EG // NBUF, pair, 0)
            swait((EG - 2) % NBUF, b, EG - 2)
            swait((EG - 1) % NBUF, b, EG - 1)

        idx_fetch(0, 0)
        idx_wait(0)

        def gpair(q, _):
            for b in range(2):
                g = q * 2 + b

                @pl.when(g + 1 < NG)
                def _():
                    idx_fetch(g + 1, 1 - b)

                ring(g, b)

                @pl.when(g + 1 < NG)
                def _():
                    idx_wait(1 - b)
            return 0
        lax.fori_loop(0, NG // 2, gpair, 0)

    @pl.when(c == 0)
    def _():
        run(hn0_hbm)

    @pl.when(c == 1)
    def _():
        run(hn1_hbm)

    plsc.subcore_barrier()

    def drain(out_hbm):
        for r in range(RT // CHUNK):
            slot = r % 2
            if r >= 2:
                pltpu.make_async_copy(
                    stg[slot],
                    out_hbm.at[pl.ds(s * RT + (r - 2) * CHUNK, CHUNK)],
                    ssem[slot]).wait()
            pltpu.sync_copy(acc_sp.at[pl.ds(s * RT + r * CHUNK, CHUNK)],
                            stg[slot])
            pltpu.async_copy(stg[slot],
                             out_hbm.at[pl.ds(s * RT + r * CHUNK, CHUNK)],
                             ssem[slot])
        for r in (RT // CHUNK - 2, RT // CHUNK - 1):
            slot = r % 2
            pltpu.make_async_copy(
                stg[slot], out_hbm.at[pl.ds(s * RT + r * CHUNK, CHUNK)],
                ssem[slot]).wait()

    @pl.when(c == 0)
    def _():
        drain(acc0_hbm)

    @pl.when(c == 1)
    def _():
        drain(acc1_hbm)


def _msg(srcp, dst2, hn0, hn1):
    f = pl.kernel(
        _msg_body,
        out_type=[
            jax.ShapeDtypeStruct((R_PAD, HALF), jnp.float32),
            jax.ShapeDtypeStruct((R_PAD, HALF), jnp.float32),
        ],
        mesh=_sc_mesh(),
        scratch_types=[
            pltpu.VMEM((2, EG * CHUNK), jnp.int32),
            pltpu.VMEM((2, EG, CHUNK), jnp.int32),
            pltpu.VMEM((CHUNK, HALF), jnp.float32),
            pltpu.VMEM((CHUNK, HALF), jnp.float32),
            pltpu.VMEM_SHARED((R_PAD, HALF), jnp.float32),
            pltpu.SemaphoreType.DMA,
            pltpu.SemaphoreType.DMA,
            pltpu.SemaphoreType.DMA,
            pltpu.SemaphoreType.DMA,
            pltpu.SemaphoreType.DMA,
        ],
    )
    return f(srcp, dst2, hn0, hn1)


def kernel(x, edge_index, pos_embedding, Wp, Wc, bc):
    pad = E_PAD - E
    dummy = N + jnp.arange(pad, dtype=jnp.int32) % (R_PAD - N)
    srcp = jnp.concatenate([edge_index[0], dummy])
    dstp = jnp.concatenate([edge_index[1], dummy])
    dst2 = dstp.reshape(C_T, CHUNK)

    y = _proj(x, pos_embedding, Wp, Wc)
    degp = _deg(dst2)
    degT = degp.reshape(NC, R_PAD).T
    hn0, hn1 = _scale(y, degT)
    acc0, acc1 = _msg(srcp, dst2, hn0, hn1)
    bc2 = bc.reshape(2, HALF)
    return _final(acc0, acc1, hn0, hn1, degT, bc2)


# speedup vs baseline: 27.1743x; 1.0204x over previous
import functools

import jax
import jax.numpy as jnp
from jax import lax
from jax.experimental import pallas as pl
from jax.experimental.pallas import tpu as pltpu
from jax.experimental.pallas import tpu_sc as plsc

N = 10000
E = 320000
IN_CH = 256
OUT_CH = 256
POS = 8
HALF = OUT_CH // 2

NC = 2
NS = 16
L = 16

CHUNK = 128
R_PAD = 10240
C_T = 2560
E_PAD = C_T * CHUNK
CB = C_T // (NC * NS)
CD = C_T // NS
EG = 16
NG = CD // EG
NBUF = 2
RT = R_PAD // NS

BR = 2000
BRP = 2048


def _proj_body(x_ref, pos_ref, wp_ref, wc_ref, y_ref):
    x = x_ref[...]
    pos = pos_ref[...]
    acc = jnp.zeros((BR, OUT_CH), jnp.float32)
    for p in range(POS):
        t = lax.dot_general(x, wp_ref[p * OUT_CH:(p + 1) * OUT_CH, :],
                            (((1,), (1,)), ((), ())),
                            preferred_element_type=jnp.float32)
        acc = acc + pos[:, p:p + 1] * t
    y_ref[...] = lax.dot_general(acc, wc_ref[...],
                                 (((1,), (1,)), ((), ())),
                                 preferred_element_type=jnp.float32)


def _proj(x, pos, Wp, Wc):
    return pl.pallas_call(
        _proj_body,
        grid=(N // BR,),
        in_specs=[
            pl.BlockSpec((BR, IN_CH), lambda i: (i, 0)),
            pl.BlockSpec((BR, POS), lambda i: (i, 0)),
            pl.BlockSpec((POS * OUT_CH, IN_CH), lambda i: (0, 0)),
            pl.BlockSpec((OUT_CH, OUT_CH), lambda i: (0, 0)),
        ],
        out_specs=pl.BlockSpec((BR, OUT_CH), lambda i: (i, 0)),
        out_shape=jax.ShapeDtypeStruct((N, OUT_CH), jnp.float32),
    )(x, pos, Wp, Wc)


def _scale_body(y_ref, degT_ref, hn0_ref, hn1_ref):
    deg = degT_ref[:, 0:1] + degT_ref[:, 1:2] + 1.0
    dinv = lax.rsqrt(deg)
    hn = y_ref[...] * dinv
    hn0_ref[...] = hn[:, :HALF]
    hn1_ref[...] = hn[:, HALF:]


def _scale(y, degT):
    return pl.pallas_call(
        _scale_body,
        grid=(R_PAD // BRP,),
        in_specs=[
            pl.BlockSpec((BRP, OUT_CH), lambda i: (i, 0)),
            pl.BlockSpec((BRP, 2), lambda i: (i, 0)),
        ],
        out_specs=[
            pl.BlockSpec((BRP, HALF), lambda i: (i, 0)),
            pl.BlockSpec((BRP, HALF), lambda i: (i, 0)),
        ],
        out_shape=[
            jax.ShapeDtypeStruct((R_PAD, HALF), jnp.float32),
            jax.ShapeDtypeStruct((R_PAD, HALF), jnp.float32),
        ],
    )(y, degT)


def _final_body(acc0_ref, acc1_ref, hn0_ref, hn1_ref, degT_ref, bc_ref, out_ref):
    deg = degT_ref[:, 0:1] + degT_ref[:, 1:2] + 1.0
    dinv = lax.rsqrt(deg)
    lo = (acc0_ref[...] + hn0_ref[...]) * dinv + bc_ref[0:1, :]
    hi = (acc1_ref[...] + hn1_ref[...]) * dinv + bc_ref[1:2, :]
    out_ref[...] = jnp.concatenate([lo, hi], axis=1)


def _final(acc0, acc1, hn0, hn1, degT, bc2):
    return pl.pallas_call(
        _final_body,
        grid=(N // BR,),
        in_specs=[
            pl.BlockSpec((BR, HALF), lambda i: (i, 0)),
            pl.BlockSpec((BR, HALF), lambda i: (i, 0)),
            pl.BlockSpec((BR, HALF), lambda i: (i, 0)),
            pl.BlockSpec((BR, HALF), lambda i: (i, 0)),
            pl.BlockSpec((BR, 2), lambda i: (i, 0)),
            pl.BlockSpec((2, HALF), lambda i: (0, 0)),
        ],
        out_specs=pl.BlockSpec((BR, OUT_CH), lambda i: (i, 0)),
        out_shape=jax.ShapeDtypeStruct((N, OUT_CH), jnp.float32),
    )(acc0, acc1, hn0, hn1, degT, bc2)


def _sc_mesh():
    return plsc.VectorSubcoreMesh(core_axis_name="c", subcore_axis_name="s",
                                  num_cores=NC, num_subcores=NS)


def _deg_body(dst2_hbm, degp_hbm, dst2d_v, ones_v, dbuf_v, deg_sp, dsem):
    c = lax.axis_index("c")
    s = lax.axis_index("s")
    w = c * NS + s
    for i in range(CHUNK // L):
        ones_v[pl.ds(i * L, L)] = jnp.ones((L,), jnp.float32)

    def zfill(i, _):
        dbuf_v[pl.ds(i * L, L)] = jnp.zeros((L,), jnp.float32)
        return 0
    lax.fori_loop(0, RT // L, zfill, 0)
    pltpu.sync_copy(dbuf_v, deg_sp.at[pl.ds(s * RT, RT)])
    plsc.subcore_barrier()

    pltpu.sync_copy(dst2_hbm.at[pl.ds(w * CB, CB)], dst2d_v)

    DB = 8

    def body(q, _):
        for k in range(DB):
            pltpu.async_copy(ones_v, deg_sp.at[dst2d_v.at[q * DB + k]],
                             dsem, add=True)
        for k in range(DB):
            pltpu.make_async_copy(ones_v, deg_sp.at[dst2d_v.at[q * DB + k]],
                                  dsem).wait()
        return 0
    lax.fori_loop(0, CB // DB, body, 0)
    plsc.subcore_barrier()

    pltpu.sync_copy(deg_sp.at[pl.ds(s * RT, RT)], dbuf_v)
    pltpu.sync_copy(dbuf_v, degp_hbm.at[pl.ds(c * R_PAD + s * RT, RT)])


def _deg(dst2):
    f = pl.kernel(
        _deg_body,
        out_type=jax.ShapeDtypeStruct((NC * R_PAD,), jnp.float32),
        mesh=_sc_mesh(),
        scratch_types=[
            pltpu.VMEM((CB, CHUNK), jnp.int32),
            pltpu.VMEM((CHUNK,), jnp.float32),
            pltpu.VMEM((RT,), jnp.float32),
            pltpu.VMEM_SHARED((R_PAD,), jnp.float32),
            pltpu.SemaphoreType.DMA,
        ],
    )
    return f(dst2)


def _msg_body(src_hbm, dst2_hbm, hn0_hbm, hn1_hbm, acc0_hbm, acc1_hbm,
              src_v, dst2d_v, stg0, stg1, acc_sp, gs0, gs1, ss0, ss1, isem):
    c = lax.axis_index("c")
    s = lax.axis_index("s")
    stg = (stg0, stg1)
    gsem = (gs0, gs1)
    ssem = (ss0, ss1)

    def zrow(i, _):
        for k in range(CHUNK // L):
            stg0[i, pl.ds(k * L, L)] = jnp.zeros((L,), jnp.float32)
        return 0
    lax.fori_loop(0, CHUNK, zrow, 0)
    for r in range(RT // CHUNK):
        pltpu.sync_copy(stg0, acc_sp.at[pl.ds(s * RT + r * CHUNK, CHUNK)])
    plsc.subcore_barrier()

    def idx_fetch(g, b):
        base = pl.multiple_of(s * CD + g * EG, 8)
        pltpu.async_copy(src_hbm.at[pl.ds(base * CHUNK, EG * CHUNK)],
                         src_v.at[b], isem)
        pltpu.async_copy(dst2_hbm.at[pl.ds(base, EG)], dst2d_v.at[b], isem)

    def idx_wait(b):
        pltpu.make_async_copy(src_hbm.at[pl.ds(0, EG * CHUNK)],
                              src_v.at[b], isem).wait()
        pltpu.make_async_copy(dst2_hbm.at[pl.ds(0, EG)],
                              dst2d_v.at[b], isem).wait()

    def run(hn_hbm):
        def gwait(slot, b, j):
            pltpu.make_async_copy(
                hn_hbm.at[src_v.at[b].at[pl.ds(j * CHUNK, CHUNK)]],
                stg[slot], gsem[slot]).wait()

        def swait(slot, b, j):
            pltpu.make_async_copy(stg[slot], acc_sp.at[dst2d_v.at[b].at[j]],
                                  ssem[slot]).wait()

        def gissue(slot, b, j):
            pltpu.async_copy(
                hn_hbm.at[src_v.at[b].at[pl.ds(j * CHUNK, CHUNK)]],
                stg[slot], gsem[slot])

        def ring(g, b):
            gissue(0, b, 0)

            def pair(p, _):
                for slot in range(NBUF):
                    j = p * NBUF + slot
                    oslot = 1 - slot
                    gwait(slot, b, j)
                    pltpu.async_copy(stg[slot],
                                     acc_sp.at[dst2d_v.at[b].at[j]],
                                     ssem[slot], add=True)

                    @pl.when(jnp.logical_and(j > 0, j + 1 < EG))
                    def _():
                        swait(oslot, b, j - 1)
                        gissue(oslot, b, j + 1)

                    @pl.when(j == 0)
                    def _():
                        gissue(oslot, b, 1)
                return 0
            lax.fori_loop(0, EG // NBUF, pair, 0)
            swait((EG - 2) % NBUF, b, EG - 2)
            swait((EG - 1) % NBUF, b, EG - 1)

        idx_fetch(0, 0)
        idx_wait(0)

        def gpair(q, _):
            for b in range(2):
                g = q * 2 + b

                @pl.when(g + 1 < NG)
                def _():
                    idx_fetch(g + 1, 1 - b)

                ring(g, b)

                @pl.when(g + 1 < NG)
                def _():
                    idx_wait(1 - b)
            return 0
        lax.fori_loop(0, NG // 2, gpair, 0)

    @pl.when(c == 0)
    def _():
        run(hn0_hbm)

    @pl.when(c == 1)
    def _():
        run(hn1_hbm)

    plsc.subcore_barrier()

    def drain(out_hbm):
        for r in range(RT // CHUNK):
            slot = r % 2
            if r >= 2:
                pltpu.make_async_copy(
                    stg[slot],
                    out_hbm.at[pl.ds(s * RT + (r - 2) * CHUNK, CHUNK)],
                    ssem[slot]).wait()
            pltpu.sync_copy(acc_sp.at[pl.ds(s * RT + r * CHUNK, CHUNK)],
                            stg[slot])
            pltpu.async_copy(stg[slot],
                             out_hbm.at[pl.ds(s * RT + r * CHUNK, CHUNK)],
                             ssem[slot])
        for r in (RT // CHUNK - 2, RT // CHUNK - 1):
            slot = r % 2
            pltpu.make_async_copy(
                stg[slot], out_hbm.at[pl.ds(s * RT + r * CHUNK, CHUNK)],
                ssem[slot]).wait()

    @pl.when(c == 0)
    def _():
        drain(acc0_hbm)

    @pl.when(c == 1)
    def _():
        drain(acc1_hbm)


def _msg(srcp, dst2, hn0, hn1):
    f = pl.kernel(
        _msg_body,
        out_type=[
            jax.ShapeDtypeStruct((R_PAD, HALF), jnp.float32),
            jax.ShapeDtypeStruct((R_PAD, HALF), jnp.float32),
        ],
        mesh=_sc_mesh(),
        scratch_types=[
            pltpu.VMEM((2, EG * CHUNK), jnp.int32),
            pltpu.VMEM((2, EG, CHUNK), jnp.int32),
            pltpu.VMEM((CHUNK, HALF), jnp.float32),
            pltpu.VMEM((CHUNK, HALF), jnp.float32),
            pltpu.VMEM_SHARED((R_PAD, HALF), jnp.float32),
            pltpu.SemaphoreType.DMA,
            pltpu.SemaphoreType.DMA,
            pltpu.SemaphoreType.DMA,
            pltpu.SemaphoreType.DMA,
            pltpu.SemaphoreType.DMA,
        ],
    )
    return f(srcp, dst2, hn0, hn1)


def kernel(x, edge_index, pos_embedding, Wp, Wc, bc):
    pad = E_PAD - E
    dummy = N + jnp.arange(pad, dtype=jnp.int32) % (R_PAD - N)
    srcp = jnp.concatenate([edge_index[0], dummy])
    dstp = jnp.concatenate([edge_index[1], dummy])
    dst2 = dstp.reshape(C_T, CHUNK)

    y = _proj(x, pos_embedding, Wp, Wc)
    degp = _deg(dst2)
    degT = degp.reshape(NC, R_PAD).T
    hn0, hn1 = _scale(y, degT)
    acc0, acc1 = _msg(srcp, dst2, hn0, hn1)
    bc2 = bc.reshape(2, HALF)
    return _final(acc0, acc1, hn0, hn1, degT, bc2)


# speedup vs baseline: 27.1883x; 1.0005x over previous
import functools

import jax
import jax.numpy as jnp
from jax import lax
from jax.experimental import pallas as pl
from jax.experimental.pallas import tpu as pltpu
from jax.experimental.pallas import tpu_sc as plsc

N = 10000
E = 320000
IN_CH = 256
OUT_CH = 256
POS = 8
HALF = OUT_CH // 2

NC = 2
NS = 16
L = 16

CHUNK = 128
R_PAD = 10240
C_T = 2560
E_PAD = C_T * CHUNK
CB = C_T // (NC * NS)
C_REAL = E // CHUNK
CD = C_T // NS
EG = 16
NG = CD // EG
NBUF = 2
RT = R_PAD // NS

BR = 2000
BRP = 2048


def _proj_body(x_ref, pos_ref, wp_ref, wc_ref, y_ref):
    x = x_ref[...]
    pos = pos_ref[...]
    acc = jnp.zeros((BR, OUT_CH), jnp.float32)
    for p in range(POS):
        t = lax.dot_general(x, wp_ref[p * OUT_CH:(p + 1) * OUT_CH, :],
                            (((1,), (1,)), ((), ())),
                            preferred_element_type=jnp.float32)
        acc = acc + pos[:, p:p + 1] * t
    y_ref[...] = lax.dot_general(acc, wc_ref[...],
                                 (((1,), (1,)), ((), ())),
                                 preferred_element_type=jnp.float32)


def _proj(x, pos, Wp, Wc):
    return pl.pallas_call(
        _proj_body,
        grid=(N // BR,),
        in_specs=[
            pl.BlockSpec((BR, IN_CH), lambda i: (i, 0)),
            pl.BlockSpec((BR, POS), lambda i: (i, 0)),
            pl.BlockSpec((POS * OUT_CH, IN_CH), lambda i: (0, 0)),
            pl.BlockSpec((OUT_CH, OUT_CH), lambda i: (0, 0)),
        ],
        out_specs=pl.BlockSpec((BR, OUT_CH), lambda i: (i, 0)),
        out_shape=jax.ShapeDtypeStruct((N, OUT_CH), jnp.float32),
    )(x, pos, Wp, Wc)


def _scale_body(y_ref, degT_ref, hn0_ref, hn1_ref):
    deg = degT_ref[:, 0:1] + degT_ref[:, 1:2] + 1.0
    dinv = lax.rsqrt(deg)
    hn = y_ref[...] * dinv
    hn0_ref[...] = hn[:, :HALF]
    hn1_ref[...] = hn[:, HALF:]


def _scale(y, degT):
    return pl.pallas_call(
        _scale_body,
        grid=(R_PAD // BRP,),
        in_specs=[
            pl.BlockSpec((BRP, OUT_CH), lambda i: (i, 0)),
            pl.BlockSpec((BRP, 2), lambda i: (i, 0)),
        ],
        out_specs=[
            pl.BlockSpec((BRP, HALF), lambda i: (i, 0)),
            pl.BlockSpec((BRP, HALF), lambda i: (i, 0)),
        ],
        out_shape=[
            jax.ShapeDtypeStruct((R_PAD, HALF), jnp.float32),
            jax.ShapeDtypeStruct((R_PAD, HALF), jnp.float32),
        ],
    )(y, degT)


def _final_body(acc0_ref, acc1_ref, hn0_ref, hn1_ref, degT_ref, bc_ref, out_ref):
    deg = degT_ref[:, 0:1] + degT_ref[:, 1:2] + 1.0
    dinv = lax.rsqrt(deg)
    lo = (acc0_ref[...] + hn0_ref[...]) * dinv + bc_ref[0:1, :]
    hi = (acc1_ref[...] + hn1_ref[...]) * dinv + bc_ref[1:2, :]
    out_ref[...] = jnp.concatenate([lo, hi], axis=1)


def _final(acc0, acc1, hn0, hn1, degT, bc2):
    return pl.pallas_call(
        _final_body,
        grid=(N // BR,),
        in_specs=[
            pl.BlockSpec((BR, HALF), lambda i: (i, 0)),
            pl.BlockSpec((BR, HALF), lambda i: (i, 0)),
            pl.BlockSpec((BR, HALF), lambda i: (i, 0)),
            pl.BlockSpec((BR, HALF), lambda i: (i, 0)),
            pl.BlockSpec((BR, 2), lambda i: (i, 0)),
            pl.BlockSpec((2, HALF), lambda i: (0, 0)),
        ],
        out_specs=pl.BlockSpec((BR, OUT_CH), lambda i: (i, 0)),
        out_shape=jax.ShapeDtypeStruct((N, OUT_CH), jnp.float32),
    )(acc0, acc1, hn0, hn1, degT, bc2)


def _sc_mesh():
    return plsc.VectorSubcoreMesh(core_axis_name="c", subcore_axis_name="s",
                                  num_cores=NC, num_subcores=NS)


def _deg_body(dst2_hbm, degp_hbm, dst2d_v, ones_v, dbuf_v, deg_sp, dsem):
    c = lax.axis_index("c")
    s = lax.axis_index("s")
    w = c * NS + s
    for i in range(CHUNK // L):
        ones_v[pl.ds(i * L, L)] = jnp.ones((L,), jnp.float32)

    def zfill(i, _):
        dbuf_v[pl.ds(i * L, L)] = jnp.zeros((L,), jnp.float32)
        return 0
    lax.fori_loop(0, RT // L, zfill, 0)
    pltpu.sync_copy(dbuf_v, deg_sp.at[pl.ds(s * RT, RT)])
    plsc.subcore_barrier()

    pltpu.sync_copy(dst2_hbm.at[pl.ds(w * CB, CB)], dst2d_v)

    DB = 8

    def body(q, _):
        for k in range(DB):
            pltpu.async_copy(ones_v, deg_sp.at[dst2d_v.at[q * DB + k]],
                             dsem, add=True)
        for k in range(DB):
            pltpu.make_async_copy(ones_v, deg_sp.at[dst2d_v.at[q * DB + k]],
                                  dsem).wait()
        return 0
    lax.fori_loop(0, CB // DB, body, 0)
    plsc.subcore_barrier()

    pltpu.sync_copy(deg_sp.at[pl.ds(s * RT, RT)], dbuf_v)
    pltpu.sync_copy(dbuf_v, degp_hbm.at[pl.ds(c * R_PAD + s * RT, RT)])


def _deg(dst2):
    f = pl.kernel(
        _deg_body,
        out_type=jax.ShapeDtypeStruct((NC * R_PAD,), jnp.float32),
        mesh=_sc_mesh(),
        scratch_types=[
            pltpu.VMEM((CB, CHUNK), jnp.int32),
            pltpu.VMEM((CHUNK,), jnp.float32),
            pltpu.VMEM((RT,), jnp.float32),
            pltpu.VMEM_SHARED((R_PAD,), jnp.float32),
            pltpu.SemaphoreType.DMA,
        ],
    )
    return f(dst2)


def _msg_body(src_hbm, dst2_hbm, hn0_hbm, hn1_hbm, acc0_hbm, acc1_hbm,
              src_v, dst2d_v, stg0, stg1, acc_sp, gs0, gs1, ss0, ss1, isem):
    c = lax.axis_index("c")
    s = lax.axis_index("s")
    stg = (stg0, stg1)
    gsem = (gs0, gs1)
    ssem = (ss0, ss1)

    def zrow(i, _):
        for k in range(CHUNK // L):
            stg0[i, pl.ds(k * L, L)] = jnp.zeros((L,), jnp.float32)
        return 0
    lax.fori_loop(0, CHUNK, zrow, 0)
    for r in range(RT // CHUNK):
        pltpu.sync_copy(stg0, acc_sp.at[pl.ds(s * RT + r * CHUNK, CHUNK)])
    plsc.subcore_barrier()

    def idx_fetch(g, b):
        base = pl.multiple_of(s * CD + g * EG, 8)
        pltpu.async_copy(src_hbm.at[pl.ds(base * CHUNK, EG * CHUNK)],
                         src_v.at[b], isem)
        pltpu.async_copy(dst2_hbm.at[pl.ds(base, EG)], dst2d_v.at[b], isem)

    def idx_wait(b):
        pltpu.make_async_copy(src_hbm.at[pl.ds(0, EG * CHUNK)],
                              src_v.at[b], isem).wait()
        pltpu.make_async_copy(dst2_hbm.at[pl.ds(0, EG)],
                              dst2d_v.at[b], isem).wait()

    def run(hn_hbm):
        def gwait(slot, b, j):
            pltpu.make_async_copy(
                hn_hbm.at[src_v.at[b].at[pl.ds(j * CHUNK, CHUNK)]],
                stg[slot], gsem[slot]).wait()

        def swait(slot, b, j):
            pltpu.make_async_copy(stg[slot], acc_sp.at[dst2d_v.at[b].at[j]],
                                  ssem[slot]).wait()

        def gissue(slot, b, j):
            pltpu.async_copy(
                hn_hbm.at[src_v.at[b].at[pl.ds(j * CHUNK, CHUNK)]],
                stg[slot], gsem[slot])

        def ring(g, b):
            gissue(0, b, 0)

            def pair(p, _):
                for slot in range(NBUF):
                    j = p * NBUF + slot
                    oslot = 1 - slot
                    gwait(slot, b, j)
                    pltpu.async_copy(stg[slot],
                                     acc_sp.at[dst2d_v.at[b].at[j]],
                                     ssem[slot], add=True)

                    @pl.when(jnp.logical_and(j > 0, j + 1 < EG))
                    def _():
                        swait(oslot, b, j - 1)
                        gissue(oslot, b, j + 1)

                    @pl.when(j == 0)
                    def _():
                        gissue(oslot, b, 1)
                return 0
            lax.fori_loop(0, EG // NBUF, pair, 0)
            swait((EG - 2) % NBUF, b, EG - 2)
            swait((EG - 1) % NBUF, b, EG - 1)

        idx_fetch(0, 0)
        idx_wait(0)

        def gpair(q, _):
            for b in range(2):
                g = q * 2 + b

                @pl.when(g + 1 < NG)
                def _():
                    idx_fetch(g + 1, 1 - b)

                ring(g, b)

                @pl.when(g + 1 < NG)
                def _():
                    idx_wait(1 - b)
            return 0
        lax.fori_loop(0, NG // 2, gpair, 0)

    @pl.when(c == 0)
    def _():
        run(hn0_hbm)

    @pl.when(c == 1)
    def _():
        run(hn1_hbm)

    plsc.subcore_barrier()

    def drain(out_hbm):
        for r in range(RT // CHUNK):
            slot = r % 2
            if r >= 2:
                pltpu.make_async_copy(
                    stg[slot],
                    out_hbm.at[pl.ds(s * RT + (r - 2) * CHUNK, CHUNK)],
                    ssem[slot]).wait()
            pltpu.sync_copy(acc_sp.at[pl.ds(s * RT + r * CHUNK, CHUNK)],
                            stg[slot])
            pltpu.async_copy(stg[slot],
                             out_hbm.at[pl.ds(s * RT + r * CHUNK, CHUNK)],
                             ssem[slot])
        for r in (RT // CHUNK - 2, RT // CHUNK - 1):
            slot = r % 2
            pltpu.make_async_copy(
                stg[slot], out_hbm.at[pl.ds(s * RT + r * CHUNK, CHUNK)],
                ssem[slot]).wait()

    @pl.when(c == 0)
    def _():
        drain(acc0_hbm)

    @pl.when(c == 1)
    def _():
        drain(acc1_hbm)


def _msg(srcp, dst2, hn0, hn1):
    f = pl.kernel(
        _msg_body,
        out_type=[
            jax.ShapeDtypeStruct((R_PAD, HALF), jnp.float32),
            jax.ShapeDtypeStruct((R_PAD, HALF), jnp.float32),
        ],
        mesh=_sc_mesh(),
        scratch_types=[
            pltpu.VMEM((2, EG * CHUNK), jnp.int32),
            pltpu.VMEM((2, EG, CHUNK), jnp.int32),
            pltpu.VMEM((CHUNK, HALF), jnp.float32),
            pltpu.VMEM((CHUNK, HALF), jnp.float32),
            pltpu.VMEM_SHARED((R_PAD, HALF), jnp.float32),
            pltpu.SemaphoreType.DMA,
            pltpu.SemaphoreType.DMA,
            pltpu.SemaphoreType.DMA,
            pltpu.SemaphoreType.DMA,
            pltpu.SemaphoreType.DMA,
        ],
    )
    return f(srcp, dst2, hn0, hn1)


def kernel(x, edge_index, pos_embedding, Wp, Wc, bc):
    pad = E_PAD - E
    dummy = N + jnp.arange(pad, dtype=jnp.int32) % (R_PAD - N)
    srcp = jnp.concatenate([edge_index[0], dummy])
    dstp = jnp.concatenate([edge_index[1], dummy])
    dst2 = dstp.reshape(C_T, CHUNK)

    y = _proj(x, pos_embedding, Wp, Wc)
    degp = _deg(dst2)
    degT = degp.reshape(NC, R_PAD).T
    hn0, hn1 = _scale(y, degT)
    acc0, acc1 = _msg(srcp, dst2, hn0, hn1)
    bc2 = bc.reshape(2, HALF)
    return _final(acc0, acc1, hn0, hn1, degT, bc2)
